# Initial kernel scaffold; baseline (speedup 1.0000x reference)
import jax
import jax.numpy as jnp
from jax.experimental import pallas as pl


def kernel(log_probs, targets, target_lengths):
    raise NotImplementedError("write your pallas kernel here")


# speedup vs baseline: 16.9343x; 16.9343x over previous
import functools

import jax
import jax.numpy as jnp
import numpy as np
from jax.experimental import pallas as pl
from jax.experimental.pallas import tpu as pltpu

DEPTH = 8
N_NODES = 511
N_PAD = 512
T = 64
B = 128
V = 1024
LOG_EPS = -64.0
EPS = float(np.exp(-64.0))
NEG_BIG = -1e30


def _leaf_interval(depth, start_idx=0):
    if depth == 0:
        return ([start_idx], [start_idx], [], start_idx)
    l_l, l_r, l_adj, last = _leaf_interval(depth - 1, start_idx)
    my = last + 1
    r_l, r_r, r_adj, last = _leaf_interval(depth - 1, my + 1)
    return ([my] + l_l, [my] + r_r,
            [(a, b) for a in l_r for b in r_l] + l_adj + r_adj, last)


_START, _END, _ADJ, _ = _leaf_interval(DEPTH)
_OUT_IDX = np.array([i for i, _ in _ADJ], dtype=np.int32)
_IN_IDX = np.array([j for _, j in _ADJ], dtype=np.int32)
_OUT_UNIQ = np.unique(_OUT_IDX)

_M_NP = np.zeros((N_PAD, N_PAD), dtype=np.float32)
_M_NP[_OUT_IDX, _IN_IDX] = 1.0

_MASKS_NP = np.zeros((8, N_PAD), dtype=np.float32)
_MASKS_NP[0, _OUT_UNIQ] = 1.0
_MASKS_NP[1, np.array(_END, dtype=np.int32)] = 1.0
_MASKS_NP[2, np.array(_START, dtype=np.int32)] = 1.0


def _gather_kernel(tgt_ref, lp_ref, out_ref):
    gn = pl.program_id(0)
    nmask = (gn * 128 + jax.lax.broadcasted_iota(jnp.int32, (T, 128), 1)) < N_NODES
    v_iota = jax.lax.broadcasted_iota(jnp.int32, (T, V), 1)
    for b in range(8):
        oh = (v_iota == tgt_ref[0, b, :][:, None]).astype(jnp.float32)
        res = jax.lax.dot_general(
            oh, lp_ref[:, b, :],
            dimension_numbers=(((1,), (1,)), ((), ())),
            preferred_element_type=jnp.float32)
        out_ref[:, b, :] = jnp.where(nmask, res, LOG_EPS)


def _loop_kernel(elp_ref, m_ref, masks_ref, tl_ref, out_ref, prev_scr):
    out_m = masks_ref[0:1, :]
    end_m = masks_ref[1:2, :]
    start_m = masks_ref[2:3, :]
    prev_scr[...] = jnp.broadcast_to(
        jnp.where(start_m > 0, jnp.float32(0.0), LOG_EPS), (B, N_PAD))
    tl = tl_ref[...]
    mmat = m_ref[...]

    def body(t, acc):
        log_prev = prev_scr[...]
        log_curr = log_prev + elp_ref[t]
        mo = jnp.where(out_m > 0, log_curr, NEG_BIG)
        me = jnp.where(end_m > 0, log_curr, NEG_BIG)
        m1 = jnp.max(mo, axis=-1, keepdims=True)
        m2 = jnp.max(me, axis=-1, keepdims=True)
        shift = jnp.where(end_m > 0, m2, m1)
        e = jnp.exp(log_curr - shift)
        s_out = jnp.sum(e * out_m, axis=-1, keepdims=True)
        s_end = jnp.sum(e * end_m, axis=-1, keepdims=True)
        log_c = m1 + jnp.log(s_out)
        end_vals = m2 + jnp.log(s_end)
        tp1 = t + 1
        acc = acc + jnp.where(tl == tp1, end_vals, jnp.float32(0.0))
        acc = acc + jnp.where(tl > tp1, log_c, jnp.float32(0.0))
        x = log_curr - log_c
        safe = x < LOG_EPS
        a = jnp.where(safe, EPS, jnp.exp(jnp.where(safe, LOG_EPS, x)))
        prev = jnp.dot(a, mmat, preferred_element_type=jnp.float32)
        psafe = prev < EPS
        prev_scr[...] = jnp.where(
            psafe, LOG_EPS, jnp.log(jnp.where(psafe, EPS, prev)))
        return acc

    acc = jax.lax.fori_loop(0, T, body, jnp.zeros((B, 1), jnp.float32))
    out_ref[...] = -acc


@jax.jit
def kernel(log_probs, targets, target_lengths):
    tgt3 = targets.astype(jnp.int32).T.reshape(16, 8, T)
    elp = pl.pallas_call(
        _gather_kernel,
        grid=(4, 16),
        in_specs=[
            pl.BlockSpec((1, 8, T), lambda gn, gb: (gb, 0, 0)),
            pl.BlockSpec((128, 8, V), lambda gn, gb: (gn, gb, 0)),
        ],
        out_specs=pl.BlockSpec((T, 8, 128), lambda gn, gb: (0, gb, gn)),
        out_shape=jax.ShapeDtypeStruct((T, B, N_PAD), jnp.float32),
    )(tgt3, log_probs)

    mmat = jnp.asarray(_M_NP)
    masks = jnp.asarray(_MASKS_NP)
    tl = target_lengths.astype(jnp.int32).reshape(B, 1)
    neg_acc = pl.pallas_call(
        _loop_kernel,
        in_specs=[
            pl.BlockSpec((T, B, N_PAD), lambda: (0, 0, 0)),
            pl.BlockSpec((N_PAD, N_PAD), lambda: (0, 0)),
            pl.BlockSpec((8, N_PAD), lambda: (0, 0)),
            pl.BlockSpec((B, 1), lambda: (0, 0)),
        ],
        out_specs=pl.BlockSpec((B, 1), lambda: (0, 0)),
        out_shape=jax.ShapeDtypeStruct((B, 1), jnp.float32),
        scratch_shapes=[pltpu.VMEM((B, N_PAD), jnp.float32)],
        compiler_params=pltpu.CompilerParams(
            vmem_limit_bytes=100 * 1024 * 1024),
    )(elp, mmat, masks, tl)
    return neg_acc.reshape(B)
